# baseline (speedup 1.0000x reference)
import functools

import jax
import jax.numpy as jnp
from jax import lax
from jax.experimental import pallas as pl
from jax.experimental.pallas import tpu as pltpu
from jax.experimental.pallas import tpu_sc as plsc

N_NODES = 10000
D = 128
NP = 10240
E = 320000
CH = 128
N_CORES = 2
N_SUB = 16
N_W = N_CORES * N_SUB
PER_W_E = 10240
CHUNKS_PER_W = PER_W_E // CH
PER_W = PER_W_E
E_PAD = N_W * PER_W
ROWS_PER_SUB = NP // N_SUB
ZCH = 128
ZCOPIES = ROWS_PER_SUB // ZCH
DCH = 128
D_CHUNKS = PER_W_E // DCH
HW = 128


@functools.cache
def _sc_degree_kernel():
    mesh = plsc.VectorSubcoreMesh(core_axis_name="c", subcore_axis_name="s")
    return functools.partial(
        pl.kernel,
        out_type=jax.ShapeDtypeStruct((N_CORES, NP, HW), jnp.float32),
        mesh=mesh,
        scratch_types=[
            pltpu.VMEM_SHARED((NP, HW), jnp.float32),
            pltpu.VMEM((DCH, HW), jnp.float32),
            pltpu.VMEM((D_CHUNKS, DCH), jnp.int32),
        ],
    )(_sc_degree_body)


def _sc_degree_body(dst_hbm, ones_hbm, zeros_hbm, out_hbm, acc, buf, didx):
    cid = lax.axis_index("c")
    sid = lax.axis_index("s")
    pltpu.sync_copy(zeros_hbm, buf)

    @pl.loop(0, ZCOPIES)
    def _zero(i):
        pltpu.sync_copy(buf, acc.at[pl.ds(sid * ROWS_PER_SUB + i * ZCH, ZCH)])

    w = cid * N_SUB + sid
    pltpu.sync_copy(dst_hbm.at[pl.ds(w * D_CHUNKS, D_CHUNKS)], didx)
    pltpu.sync_copy(ones_hbm, buf)
    plsc.subcore_barrier()

    @pl.loop(0, D_CHUNKS)
    def _hist(g):
        pltpu.sync_copy(buf, acc.at[didx.at[g]], add=True)

    plsc.subcore_barrier()
    pltpu.sync_copy(acc.at[pl.ds(sid * ROWS_PER_SUB, ROWS_PER_SUB)],
                    out_hbm.at[cid].at[pl.ds(sid * ROWS_PER_SUB, ROWS_PER_SUB)])


NBUF = 2
N_PHASE = 2
HALF = CHUNKS_PER_W // N_PHASE


@functools.cache
def _sc_aggregate_kernel():
    mesh = plsc.VectorSubcoreMesh(core_axis_name="c", subcore_axis_name="s")
    return functools.partial(
        pl.kernel,
        out_type=jax.ShapeDtypeStruct((N_CORES, NP, D), jnp.float32),
        mesh=mesh,
        scratch_types=[
            pltpu.VMEM_SHARED((NP, D), jnp.float32),
            pltpu.VMEM((NBUF, CH, D), jnp.float32),
            pltpu.VMEM((HALF, CH), jnp.int32),
            pltpu.VMEM((HALF, CH), jnp.int32),
        ] + [pltpu.SemaphoreType.DMA] * NBUF,
    )(_sc_aggregate_body)


def _sc_aggregate_body(hp_hbm, src_hbm, dst_hbm, zeros_hbm, out_hbm,
                       acc, rows, sidx, didx, *sems):
    cid = lax.axis_index("c")
    sid = lax.axis_index("s")
    pltpu.sync_copy(zeros_hbm, rows.at[0])

    @pl.loop(0, ROWS_PER_SUB // CH)
    def _zero(i):
        pltpu.sync_copy(rows.at[0], acc.at[pl.ds(sid * ROWS_PER_SUB + i * CH, CH)])

    w = cid * N_SUB + sid
    plsc.subcore_barrier()

    def _gather(g, b):
        pltpu.make_async_copy(hp_hbm.at[sidx.at[g]], rows.at[b], sems[b]).start()

    def _drain_and_scatter(g, b):
        pltpu.make_async_copy(hp_hbm.at[sidx.at[g]], rows.at[b], sems[b]).wait()
        pltpu.sync_copy(rows.at[b], acc.at[didx.at[g]], add=True)

    for phase in range(N_PHASE):
        base = w * CHUNKS_PER_W + phase * HALF
        pltpu.sync_copy(src_hbm.at[pl.ds(base, HALF)], sidx)
        pltpu.sync_copy(dst_hbm.at[pl.ds(base, HALF)], didx)
        for b in range(NBUF):
            _gather(b, b)

        @pl.loop(0, (HALF - NBUF) // NBUF)
        def _agg(i):
            for b in range(NBUF):
                g = i * NBUF + b
                _drain_and_scatter(g, b)
                _gather(g + NBUF, b)

        for b in range(NBUF):
            _drain_and_scatter(HALF - NBUF + b, b)

    plsc.subcore_barrier()
    pltpu.sync_copy(acc.at[pl.ds(sid * ROWS_PER_SUB, ROWS_PER_SUB)],
                    out_hbm.at[cid].at[pl.ds(sid * ROWS_PER_SUB, ROWS_PER_SUB)])


BLK = 512
GRID = NP // BLK


def _mm_body(x_ref, w_ref, o_ref):
    o_ref[...] = jnp.dot(x_ref[...], w_ref[...],
                         preferred_element_type=jnp.float32,
                         precision=lax.Precision.HIGHEST)


_mm = pl.pallas_call(
    _mm_body,
    grid=(GRID,),
    in_specs=[pl.BlockSpec((BLK, D), lambda i: (i, 0)),
              pl.BlockSpec((D, D), lambda i: (0, 0))],
    out_specs=pl.BlockSpec((BLK, D), lambda i: (i, 0)),
    out_shape=jax.ShapeDtypeStruct((NP, D), jnp.float32),
)


def _dinv(deg_ref):
    deg = deg_ref[0, :, :1] + deg_ref[1, :, :1] + 1.0
    return lax.rsqrt(deg)


def _scale_body(deg_ref, h_ref, o_ref):
    o_ref[...] = h_ref[...] * _dinv(deg_ref)


_scale = pl.pallas_call(
    _scale_body,
    grid=(GRID,),
    in_specs=[pl.BlockSpec((N_CORES, BLK, HW), lambda i: (0, i, 0)),
              pl.BlockSpec((BLK, D), lambda i: (i, 0))],
    out_specs=pl.BlockSpec((BLK, D), lambda i: (i, 0)),
    out_shape=jax.ShapeDtypeStruct((NP, D), jnp.float32),
)


def _mid_body(deg_ref, agg_ref, hp_ref, b1_ref, w2_ref, o_ref):
    dinv = _dinv(deg_ref)
    z = (agg_ref[0] + agg_ref[1] + hp_ref[...]) * dinv + b1_ref[...]
    z = jnp.maximum(z, 0.0)
    h2 = jnp.dot(z, w2_ref[...], preferred_element_type=jnp.float32,
                 precision=lax.Precision.HIGHEST)
    o_ref[...] = h2 * dinv


_mid = pl.pallas_call(
    _mid_body,
    grid=(GRID,),
    in_specs=[pl.BlockSpec((N_CORES, BLK, HW), lambda i: (0, i, 0)),
              pl.BlockSpec((N_CORES, BLK, D), lambda i: (0, i, 0)),
              pl.BlockSpec((BLK, D), lambda i: (i, 0)),
              pl.BlockSpec((1, D), lambda i: (0, 0)),
              pl.BlockSpec((D, D), lambda i: (0, 0))],
    out_specs=pl.BlockSpec((BLK, D), lambda i: (i, 0)),
    out_shape=jax.ShapeDtypeStruct((NP, D), jnp.float32),
)


def _final_body(deg_ref, agg_ref, hp_ref, b2_ref, o_ref):
    dinv = _dinv(deg_ref)
    o_ref[...] = (agg_ref[0] + agg_ref[1] + hp_ref[...]) * dinv + b2_ref[...]


_final = pl.pallas_call(
    _final_body,
    grid=(GRID,),
    in_specs=[pl.BlockSpec((N_CORES, BLK, HW), lambda i: (0, i, 0)),
              pl.BlockSpec((N_CORES, BLK, D), lambda i: (0, i, 0)),
              pl.BlockSpec((BLK, D), lambda i: (i, 0)),
              pl.BlockSpec((1, D), lambda i: (0, 0))],
    out_specs=pl.BlockSpec((BLK, D), lambda i: (i, 0)),
    out_shape=jax.ShapeDtypeStruct((N_NODES, D), jnp.float32),
)


def kernel(x, edge_index, W1, b1, W2, b2):
    src = edge_index[0].astype(jnp.int32)
    dst = edge_index[1].astype(jnp.int32)
    pad = N_NODES + (jnp.arange(E_PAD - E, dtype=jnp.int32) % (NP - N_NODES))
    src_all = jnp.concatenate([src, pad])
    dst_all = jnp.concatenate([dst, pad])
    src_p = src_all.reshape(E_PAD // CH, CH)
    dst_p = dst_all.reshape(E_PAD // CH, CH)
    dst_pd = dst_all.reshape(E_PAD // DCH, DCH)
    x_p = jnp.zeros((NP, D), jnp.float32).at[:N_NODES].set(x)
    zeros_rows = jnp.zeros((CH, D), jnp.float32)
    zeros_h = jnp.zeros((DCH, HW), jnp.float32)
    ones_h = jnp.ones((DCH, HW), jnp.float32)
    b1r = b1.reshape(1, D)
    b2r = b2.reshape(1, D)

    sc_degree = _sc_degree_kernel()
    sc_aggregate = _sc_aggregate_kernel()
    degp = sc_degree(dst_pd, ones_h, zeros_h)
    h1 = _mm(x_p, W1)
    h1p = _scale(degp, h1)
    agg1 = sc_aggregate(h1p, src_p, dst_p, zeros_rows)
    h2p = _mid(degp, agg1, h1p, b1r, W2)
    agg2 = sc_aggregate(h2p, src_p, dst_p, zeros_rows)
    return _final(degp, agg2, h2p, b2r)


# speedup vs baseline: 1.1337x; 1.1337x over previous
import dataclasses
import functools

import jax
import jax.numpy as jnp
from jax import lax
from jax.experimental import pallas as pl
from jax.experimental.pallas import tpu as pltpu
from jax.experimental.pallas import tpu_sc as plsc

N_NODES = 10000
D = 128
NP = 10240
E = 320000
CH = 128
N_CORES = 2
N_SUB = 16
N_W = N_CORES * N_SUB
PER_W_E = 10240
CHUNKS_PER_W = PER_W_E // CH
PER_W = PER_W_E
E_PAD = N_W * PER_W
ROWS_PER_SUB = NP // N_SUB
ZCH = 128
ZCOPIES = ROWS_PER_SUB // ZCH
DCH = 128
D_CHUNKS = PER_W_E // DCH
HW = 128


@functools.cache
def _sc_degree_kernel():
    mesh = plsc.VectorSubcoreMesh(core_axis_name="c", subcore_axis_name="s")
    cp = pltpu.CompilerParams()
    if "needs_layout_passes" in pltpu.CompilerParams.__dataclass_fields__:
        cp = dataclasses.replace(cp, needs_layout_passes=False)
    return functools.partial(
        pl.kernel,
        out_type=jax.ShapeDtypeStruct((N_CORES, N_SUB, ROWS_PER_SUB),
                                      jnp.float32),
        mesh=mesh,
        compiler_params=cp,
        scratch_types=[
            pltpu.VMEM((NP,), jnp.float32),
            pltpu.VMEM((N_SUB, ROWS_PER_SUB), jnp.float32),
            pltpu.VMEM((D_CHUNKS, DCH), jnp.int32),
            pltpu.VMEM_SHARED((N_SUB, N_SUB, ROWS_PER_SUB), jnp.float32),
            pltpu.SemaphoreType.DMA,
        ],
    )(_sc_degree_body)


def _sc_degree_body(dst_hbm, out_hbm, hist, sumbuf, didx, shared, sem):
    cid = lax.axis_index("c")
    sid = lax.axis_index("s")
    w = cid * N_SUB + sid
    pltpu.sync_copy(dst_hbm.at[pl.ds(w * D_CHUNKS, D_CHUNKS)], didx)

    zeros16 = jnp.zeros((16,), jnp.float32)
    ones16 = jnp.ones((16,), jnp.float32)

    @pl.loop(0, NP // 16)
    def _zero(i):
        hist[pl.ds(i * 16, 16)] = zeros16

    @pl.loop(0, D_CHUNKS)
    def _chunk(g):
        @pl.loop(0, DCH // 16)
        def _grp(j):
            idx = didx[g, pl.ds(j * 16, 16)]
            plsc.addupdate_scatter(hist, [idx], ones16)

    for s in range(N_SUB):
        pltpu.make_async_copy(hist.at[pl.ds(s * ROWS_PER_SUB, ROWS_PER_SUB)],
                              shared.at[s].at[sid], sem).start()
    for s in range(N_SUB):
        pltpu.make_async_copy(hist.at[pl.ds(s * ROWS_PER_SUB, ROWS_PER_SUB)],
                              shared.at[s].at[sid], sem).wait()
    plsc.subcore_barrier()
    pltpu.sync_copy(shared.at[sid], sumbuf)

    @pl.loop(0, ROWS_PER_SUB // 16)
    def _sum(k):
        col = pl.ds(k * 16, 16)
        v = sumbuf[0, col]
        for t in range(1, N_SUB):
            v = v + sumbuf[t, col]
        hist[col] = v

    pltpu.sync_copy(hist.at[pl.ds(0, ROWS_PER_SUB)], out_hbm.at[cid].at[sid])


NBUF = 2
N_PHASE = 2
HALF = CHUNKS_PER_W // N_PHASE


@functools.cache
def _sc_aggregate_kernel():
    mesh = plsc.VectorSubcoreMesh(core_axis_name="c", subcore_axis_name="s")
    return functools.partial(
        pl.kernel,
        out_type=jax.ShapeDtypeStruct((N_CORES, NP, D), jnp.float32),
        mesh=mesh,
        scratch_types=[
            pltpu.VMEM_SHARED((NP, D), jnp.float32),
            pltpu.VMEM((NBUF, CH, D), jnp.float32),
            pltpu.VMEM((HALF, CH), jnp.int32),
            pltpu.VMEM((HALF, CH), jnp.int32),
        ] + [pltpu.SemaphoreType.DMA] * NBUF,
    )(_sc_aggregate_body)


def _sc_aggregate_body(hp_hbm, src_hbm, dst_hbm, zeros_hbm, out_hbm,
                       acc, rows, sidx, didx, *sems):
    cid = lax.axis_index("c")
    sid = lax.axis_index("s")
    pltpu.sync_copy(zeros_hbm, rows.at[0])

    @pl.loop(0, ROWS_PER_SUB // CH)
    def _zero(i):
        pltpu.sync_copy(rows.at[0], acc.at[pl.ds(sid * ROWS_PER_SUB + i * CH, CH)])

    w = cid * N_SUB + sid
    plsc.subcore_barrier()

    def _gather(g, b):
        pltpu.make_async_copy(hp_hbm.at[sidx.at[g]], rows.at[b], sems[b]).start()

    def _drain_and_scatter(g, b):
        pltpu.make_async_copy(hp_hbm.at[sidx.at[g]], rows.at[b], sems[b]).wait()
        pltpu.sync_copy(rows.at[b], acc.at[didx.at[g]], add=True)

    for phase in range(N_PHASE):
        base = w * CHUNKS_PER_W + phase * HALF
        pltpu.sync_copy(src_hbm.at[pl.ds(base, HALF)], sidx)
        pltpu.sync_copy(dst_hbm.at[pl.ds(base, HALF)], didx)
        for b in range(NBUF):
            _gather(b, b)

        @pl.loop(0, (HALF - NBUF) // NBUF)
        def _agg(i):
            for b in range(NBUF):
                g = i * NBUF + b
                _drain_and_scatter(g, b)
                _gather(g + NBUF, b)

        for b in range(NBUF):
            _drain_and_scatter(HALF - NBUF + b, b)

    plsc.subcore_barrier()
    pltpu.sync_copy(acc.at[pl.ds(sid * ROWS_PER_SUB, ROWS_PER_SUB)],
                    out_hbm.at[cid].at[pl.ds(sid * ROWS_PER_SUB, ROWS_PER_SUB)])


BLK = 512
GRID = NP // BLK


def _mm_body(x_ref, w_ref, o_ref):
    o_ref[...] = jnp.dot(x_ref[...], w_ref[...],
                         preferred_element_type=jnp.float32,
                         precision=lax.Precision.HIGHEST)


_mm = pl.pallas_call(
    _mm_body,
    grid=(GRID,),
    in_specs=[pl.BlockSpec((BLK, D), lambda i: (i, 0)),
              pl.BlockSpec((D, D), lambda i: (0, 0))],
    out_specs=pl.BlockSpec((BLK, D), lambda i: (i, 0)),
    out_shape=jax.ShapeDtypeStruct((NP, D), jnp.float32),
)


def _dinv(deg_ref):
    deg = deg_ref[0] + deg_ref[1] + 1.0
    return lax.rsqrt(deg)


def _scale_body(deg_ref, h_ref, o_ref):
    o_ref[...] = h_ref[...] * _dinv(deg_ref)


_scale = pl.pallas_call(
    _scale_body,
    grid=(GRID,),
    in_specs=[pl.BlockSpec((N_CORES, BLK, 1), lambda i: (0, i, 0)),
              pl.BlockSpec((BLK, D), lambda i: (i, 0))],
    out_specs=pl.BlockSpec((BLK, D), lambda i: (i, 0)),
    out_shape=jax.ShapeDtypeStruct((NP, D), jnp.float32),
)


def _mid_body(deg_ref, agg_ref, hp_ref, b1_ref, w2_ref, o_ref):
    dinv = _dinv(deg_ref)
    z = (agg_ref[0] + agg_ref[1] + hp_ref[...]) * dinv + b1_ref[...]
    z = jnp.maximum(z, 0.0)
    h2 = jnp.dot(z, w2_ref[...], preferred_element_type=jnp.float32,
                 precision=lax.Precision.HIGHEST)
    o_ref[...] = h2 * dinv


_mid = pl.pallas_call(
    _mid_body,
    grid=(GRID,),
    in_specs=[pl.BlockSpec((N_CORES, BLK, 1), lambda i: (0, i, 0)),
              pl.BlockSpec((N_CORES, BLK, D), lambda i: (0, i, 0)),
              pl.BlockSpec((BLK, D), lambda i: (i, 0)),
              pl.BlockSpec((1, D), lambda i: (0, 0)),
              pl.BlockSpec((D, D), lambda i: (0, 0))],
    out_specs=pl.BlockSpec((BLK, D), lambda i: (i, 0)),
    out_shape=jax.ShapeDtypeStruct((NP, D), jnp.float32),
)


def _final_body(deg_ref, agg_ref, hp_ref, b2_ref, o_ref):
    dinv = _dinv(deg_ref)
    o_ref[...] = (agg_ref[0] + agg_ref[1] + hp_ref[...]) * dinv + b2_ref[...]


_final = pl.pallas_call(
    _final_body,
    grid=(GRID,),
    in_specs=[pl.BlockSpec((N_CORES, BLK, 1), lambda i: (0, i, 0)),
              pl.BlockSpec((N_CORES, BLK, D), lambda i: (0, i, 0)),
              pl.BlockSpec((BLK, D), lambda i: (i, 0)),
              pl.BlockSpec((1, D), lambda i: (0, 0))],
    out_specs=pl.BlockSpec((BLK, D), lambda i: (i, 0)),
    out_shape=jax.ShapeDtypeStruct((N_NODES, D), jnp.float32),
)


def kernel(x, edge_index, W1, b1, W2, b2):
    src = edge_index[0].astype(jnp.int32)
    dst = edge_index[1].astype(jnp.int32)
    pad = N_NODES + (jnp.arange(E_PAD - E, dtype=jnp.int32) % (NP - N_NODES))
    src_all = jnp.concatenate([src, pad])
    dst_all = jnp.concatenate([dst, pad])
    src_p = src_all.reshape(E_PAD // CH, CH)
    dst_p = dst_all.reshape(E_PAD // CH, CH)
    dst_pd = dst_all.reshape(E_PAD // DCH, DCH)
    x_p = jnp.zeros((NP, D), jnp.float32).at[:N_NODES].set(x)
    zeros_rows = jnp.zeros((CH, D), jnp.float32)
    b1r = b1.reshape(1, D)
    b2r = b2.reshape(1, D)

    sc_degree = _sc_degree_kernel()
    sc_aggregate = _sc_aggregate_kernel()
    degp = sc_degree(dst_pd).reshape(N_CORES, NP, 1)
    h1 = _mm(x_p, W1)
    h1p = _scale(degp, h1)
    agg1 = sc_aggregate(h1p, src_p, dst_p, zeros_rows)
    h2p = _mid(degp, agg1, h1p, b1r, W2)
    agg2 = sc_aggregate(h2p, src_p, dst_p, zeros_rows)
    return _final(degp, agg2, h2p, b2r)
